# baseline (speedup 1.0000x reference)
import functools

import jax
import jax.numpy as jnp
from jax.experimental import pallas as pl
from jax.experimental.pallas import tpu as pltpu

_LEVEL_PAD = 8
_BB = 2


def _quantize(x, vpad_ref):
    zeros = jnp.zeros_like(x)
    v0 = vpad_ref[:, 0:1]
    best_d = jnp.abs(x - v0)
    best_i = jnp.zeros_like(x, dtype=jnp.int32)
    best_v = v0 + zeros
    for k in range(1, _LEVEL_PAD):
        vk = vpad_ref[:, k:k + 1]
        d = jnp.abs(x - vk)
        better = d < best_d
        best_d = jnp.where(better, d, best_d)
        best_i = jnp.where(better, k, best_i)
        best_v = jnp.where(better, vk + zeros, best_v)
    return best_i, best_v


def _body(z_ref, wint_ref, bin_ref, wout_ref, bout_ref, vpad_ref,
          out_ref, idx_ref, loss_ref):
    partial = None
    for j in range(_BB):
        zb = z_ref[j]
        x = jax.lax.dot_general(wint_ref[...], zb, (((1,), (0,)), ((), ())),
                                preferred_element_type=jnp.float32)
        x = x + bin_ref[...]
        best_i, best_v = _quantize(x, vpad_ref)
        r = best_v - x
        p = jnp.sum(r * r)
        partial = p if partial is None else partial + p
        y = jax.lax.dot_general(wout_ref[...], best_v,
                                (((0,), (0,)), ((), ())),
                                preferred_element_type=jnp.float32)
        out_ref[j] = y + bout_ref[...]
        idx_ref[j] = best_i.T
    loss_ref[0] = partial[None, None]


@functools.partial(jax.jit, static_argnames=())
def kernel(z, Win, bin_, Wout, bout, v0, v1, v2, v3, v4):
    B, D, H, W = z.shape
    N = H * W
    vals = [v0, v1, v2, v3, v4]
    cd = len(vals)
    zr = z.reshape(B, D, N)
    WinT = Win.T
    big = jnp.float32(1e30)
    vpad = jnp.stack([
        jnp.concatenate([v, jnp.full((_LEVEL_PAD - v.shape[0],), big,
                                     dtype=jnp.float32)]) if v.shape[0] < _LEVEL_PAD else v
        for v in vals
    ])
    bin2 = bin_.reshape(cd, 1)
    bout2 = bout.reshape(D, 1)
    G = B // _BB

    out3, idx3, partials = pl.pallas_call(
        _body,
        grid=(G,),
        in_specs=[
            pl.BlockSpec((_BB, D, N), lambda b: (b, 0, 0)),
            pl.BlockSpec((cd, D), lambda b: (0, 0)),
            pl.BlockSpec((cd, 1), lambda b: (0, 0)),
            pl.BlockSpec((cd, D), lambda b: (0, 0)),
            pl.BlockSpec((D, 1), lambda b: (0, 0)),
            pl.BlockSpec((cd, _LEVEL_PAD), lambda b: (0, 0)),
        ],
        out_specs=[
            pl.BlockSpec((_BB, D, N), lambda b: (b, 0, 0)),
            pl.BlockSpec((_BB, N, cd), lambda b: (b, 0, 0)),
            pl.BlockSpec((1, 1, 1), lambda b: (b, 0, 0)),
        ],
        out_shape=[
            jax.ShapeDtypeStruct((B, D, N), jnp.float32),
            jax.ShapeDtypeStruct((B, N, cd), jnp.int32),
            jax.ShapeDtypeStruct((G, 1, 1), jnp.float32),
        ],
        compiler_params=pltpu.CompilerParams(
            dimension_semantics=("parallel",)),
    )(zr, WinT, bin2, Wout, bout2, vpad)

    out = out3.reshape(B, D, H, W)
    indices = idx3.reshape(B, H, W, cd)
    loss = (0.2 / (B * N * cd)) * jnp.sum(partials)
    return out, indices, loss


# speedup vs baseline: 1.0644x; 1.0644x over previous
import functools

import jax
import jax.numpy as jnp
from jax.experimental import pallas as pl
from jax.experimental.pallas import tpu as pltpu

_LEVEL_PAD = 8
_BB = 4


def _quantize(x, vpad_ref):
    zeros = jnp.zeros_like(x)
    v0 = vpad_ref[:, 0:1]
    best_d = jnp.abs(x - v0)
    best_i = jnp.zeros_like(x, dtype=jnp.int32)
    best_v = v0 + zeros
    for k in range(1, _LEVEL_PAD):
        vk = vpad_ref[:, k:k + 1]
        d = jnp.abs(x - vk)
        better = d < best_d
        best_d = jnp.where(better, d, best_d)
        best_i = jnp.where(better, k, best_i)
        best_v = jnp.where(better, vk + zeros, best_v)
    return best_i, best_v


def _body(z_ref, wint_ref, bin_ref, wout_ref, bout_ref, vpad_ref,
          out_ref, idx_ref, loss_ref):
    partial = None
    for j in range(_BB):
        zb = z_ref[j]
        x = jax.lax.dot_general(wint_ref[...], zb, (((1,), (0,)), ((), ())),
                                preferred_element_type=jnp.float32)
        x = x + bin_ref[...]
        best_i, best_v = _quantize(x, vpad_ref)
        r = best_v - x
        p = jnp.sum(r * r)
        partial = p if partial is None else partial + p
        y = jax.lax.dot_general(wout_ref[...], best_v,
                                (((0,), (0,)), ((), ())),
                                preferred_element_type=jnp.float32)
        out_ref[j] = y + bout_ref[...]
        idx_ref[j] = best_i
    loss_ref[0] = partial[None, None]


@functools.partial(jax.jit, static_argnames=())
def kernel(z, Win, bin_, Wout, bout, v0, v1, v2, v3, v4):
    B, D, H, W = z.shape
    N = H * W
    vals = [v0, v1, v2, v3, v4]
    cd = len(vals)
    zr = z.reshape(B, D, N)
    WinT = Win.T
    big = jnp.float32(1e30)
    vpad = jnp.stack([
        jnp.concatenate([v, jnp.full((_LEVEL_PAD - v.shape[0],), big,
                                     dtype=jnp.float32)]) if v.shape[0] < _LEVEL_PAD else v
        for v in vals
    ])
    bin2 = bin_.reshape(cd, 1)
    bout2 = bout.reshape(D, 1)
    G = B // _BB

    out3, idx3, partials = pl.pallas_call(
        _body,
        grid=(G,),
        in_specs=[
            pl.BlockSpec((_BB, D, N), lambda b: (b, 0, 0)),
            pl.BlockSpec((cd, D), lambda b: (0, 0)),
            pl.BlockSpec((cd, 1), lambda b: (0, 0)),
            pl.BlockSpec((cd, D), lambda b: (0, 0)),
            pl.BlockSpec((D, 1), lambda b: (0, 0)),
            pl.BlockSpec((cd, _LEVEL_PAD), lambda b: (0, 0)),
        ],
        out_specs=[
            pl.BlockSpec((_BB, D, N), lambda b: (b, 0, 0)),
            pl.BlockSpec((_BB, cd, N), lambda b: (b, 0, 0)),
            pl.BlockSpec((1, 1, 1), lambda b: (b, 0, 0)),
        ],
        out_shape=[
            jax.ShapeDtypeStruct((B, D, N), jnp.float32),
            jax.ShapeDtypeStruct((B, cd, N), jnp.int32),
            jax.ShapeDtypeStruct((G, 1, 1), jnp.float32),
        ],
        compiler_params=pltpu.CompilerParams(
            dimension_semantics=("parallel",)),
    )(zr, WinT, bin2, Wout, bout2, vpad)

    out = out3.reshape(B, D, H, W)
    indices = idx3.transpose(0, 2, 1).reshape(B, H, W, cd)
    loss = (0.2 / (B * N * cd)) * jnp.sum(partials)
    return out, indices, loss
